# initial kernel scaffold (unmeasured)
import jax
import jax.numpy as jnp
from jax import lax
from jax.experimental import pallas as pl
from jax.experimental.pallas import tpu as pltpu


def kernel(
    x,
):
    def body(*refs):
        pass

    out_shape = jax.ShapeDtypeStruct(..., jnp.float32)
    return pl.pallas_call(body, out_shape=out_shape)(...)


# baseline (device time: 181533 ns/iter reference)
import jax
import jax.numpy as jnp
from jax import lax
from jax.experimental import pallas as pl
from jax.experimental.pallas import tpu as pltpu

N_DEV = 4
M, N = 4096, 2048
HALF = M // 2
CHUNK = HALF // N_DEV


def kernel(x):

    def body(x_ref, out_ref, comm_cw, comm_ccw,
             send_cw, recv_cw, send_ccw, recv_ccw):
        my = lax.axis_index("i")
        right = lax.rem(my + 1, N_DEV)
        left = lax.rem(my + N_DEV - 1, N_DEV)

        barrier = pltpu.get_barrier_semaphore()
        for nbr in (left, right):
            pl.semaphore_signal(
                barrier, inc=1,
                device_id=(nbr,), device_id_type=pl.DeviceIdType.MESH,
            )
        pl.semaphore_wait(barrier, 2)

        out_ref[...] = x_ref[0].astype(jnp.bfloat16)

        def rows(base, c):
            return pl.ds(base + c * CHUNK, CHUNK)

        rings = (
            (1, 0, comm_cw, send_cw, recv_cw, right),
            (-1, HALF, comm_ccw, send_ccw, recv_ccw, left),
        )

        for s in range(N_DEV - 1):
            rdmas = []
            for dirn, base, comm, ssem, rsem, tgt in rings:
                send_c = lax.rem(my - dirn * s + 2 * N_DEV, N_DEV)
                rdma = pltpu.make_async_remote_copy(
                    src_ref=out_ref.at[rows(base, send_c), :],
                    dst_ref=comm.at[s],
                    send_sem=ssem.at[s],
                    recv_sem=rsem.at[s],
                    device_id=(tgt,),
                    device_id_type=pl.DeviceIdType.MESH,
                )
                rdma.start()
                rdmas.append(rdma)
            for rdma in rdmas:
                rdma.wait()
            for dirn, base, comm, ssem, rsem, tgt in rings:
                recv_c = lax.rem(my - dirn * (s + 1) + 2 * N_DEV, N_DEV)
                r = rows(base, recv_c)
                out_ref[r, :] = out_ref[r, :] + comm[s]

        for t in range(N_DEV - 1):
            h = (N_DEV - 1) + t
            rdmas = []
            for dirn, base, comm, ssem, rsem, tgt in rings:
                send_c = lax.rem(my + dirn * (1 - t) + 2 * N_DEV, N_DEV)
                rdma = pltpu.make_async_remote_copy(
                    src_ref=out_ref.at[rows(base, send_c), :],
                    dst_ref=out_ref.at[rows(base, send_c), :],
                    send_sem=ssem.at[h],
                    recv_sem=rsem.at[h],
                    device_id=(tgt,),
                    device_id_type=pl.DeviceIdType.MESH,
                )
                rdma.start()
                rdmas.append(rdma)
            for rdma in rdmas:
                rdma.wait()

    return pl.pallas_call(
        body,
        out_shape=jax.ShapeDtypeStruct((M, N), jnp.bfloat16),
        in_specs=[pl.BlockSpec(memory_space=pltpu.VMEM)],
        out_specs=pl.BlockSpec(memory_space=pltpu.VMEM),
        scratch_shapes=[
            pltpu.VMEM((N_DEV - 1, CHUNK, N), jnp.bfloat16),
            pltpu.VMEM((N_DEV - 1, CHUNK, N), jnp.bfloat16),
            pltpu.SemaphoreType.DMA((2 * (N_DEV - 1),)),
            pltpu.SemaphoreType.DMA((2 * (N_DEV - 1),)),
            pltpu.SemaphoreType.DMA((2 * (N_DEV - 1),)),
            pltpu.SemaphoreType.DMA((2 * (N_DEV - 1),)),
        ],
        compiler_params=pltpu.CompilerParams(
            collective_id=0,
            vmem_limit_bytes=64 * 1024 * 1024,
        ),
    )(x)


# device time: 170539 ns/iter; 1.0645x vs baseline; 1.0645x over previous
import jax
import jax.numpy as jnp
from jax import lax
from jax.experimental import pallas as pl
from jax.experimental.pallas import tpu as pltpu

N_DEV = 4
M, N = 4096, 2048
HALF = M // 2
CHUNK = HALF // N_DEV
STREAMS = 2
SUB = CHUNK // STREAMS
N_HOPS = 2 * (N_DEV - 1)


def kernel(x):

    def body(x_ref, out_ref, comm_cw, comm_ccw,
             ssem_cw, rsem_cw, ssem_ccw, rsem_ccw):
        my = lax.axis_index("i")
        right = lax.rem(my + 1, N_DEV)
        left = lax.rem(my + N_DEV - 1, N_DEV)

        barrier = pltpu.get_barrier_semaphore()
        for nbr in (left, right):
            pl.semaphore_signal(
                barrier, inc=1,
                device_id=(nbr,), device_id_type=pl.DeviceIdType.MESH,
            )
        pl.semaphore_wait(barrier, 2)

        out_ref[...] = x_ref[0].astype(jnp.bfloat16)

        def subrows(base, c, j):
            return pl.ds(base + c * CHUNK + j * SUB, SUB)

        rings = (
            (1, 0, comm_cw, ssem_cw, rsem_cw, right),
            (-1, HALF, comm_ccw, ssem_ccw, rsem_ccw, left),
        )

        def rs_send(ring, j, s):
            dirn, base, comm, ssem, rsem, tgt = ring
            c = lax.rem(my - dirn * s + 2 * N_DEV, N_DEV)
            rdma = pltpu.make_async_remote_copy(
                src_ref=out_ref.at[subrows(base, c, j), :],
                dst_ref=comm.at[j, s],
                send_sem=ssem.at[j, s],
                recv_sem=rsem.at[j, s],
                device_id=(tgt,),
                device_id_type=pl.DeviceIdType.MESH,
            )
            rdma.start()
            return rdma

        def ag_send(ring, j, t):
            dirn, base, comm, ssem, rsem, tgt = ring
            h = (N_DEV - 1) + t
            c = lax.rem(my + dirn * (1 - t) + 2 * N_DEV, N_DEV)
            rdma = pltpu.make_async_remote_copy(
                src_ref=out_ref.at[subrows(base, c, j), :],
                dst_ref=out_ref.at[subrows(base, c, j), :],
                send_sem=ssem.at[j, h],
                recv_sem=rsem.at[j, h],
                device_id=(tgt,),
                device_id_type=pl.DeviceIdType.MESH,
            )
            rdma.start()
            return rdma

        rdmas = {}
        for ri, ring in enumerate(rings):
            for j in range(STREAMS):
                rdmas[(ri, j, 0)] = rs_send(ring, j, 0)

        for s in range(N_DEV - 1):
            for j in range(STREAMS):
                for ri, ring in enumerate(rings):
                    dirn, base, comm, _, _, _ = ring
                    rdmas[(ri, j, s)].wait_recv()
                    c = lax.rem(my - dirn * (s + 1) + 2 * N_DEV, N_DEV)
                    r = subrows(base, c, j)
                    out_ref[r, :] = out_ref[r, :] + comm[j, s]
                    if s < N_DEV - 2:
                        rdmas[(ri, j, s + 1)] = rs_send(ring, j, s + 1)
                    else:
                        rdmas[(ri, j, s + 1)] = ag_send(ring, j, 0)

        for t in range(N_DEV - 1):
            h = (N_DEV - 1) + t
            for j in range(STREAMS):
                for ri, ring in enumerate(rings):
                    rdmas[(ri, j, h)].wait_recv()
                    if t < N_DEV - 2:
                        rdmas[(ri, j, h + 1)] = ag_send(ring, j, t + 1)

        for rdma in rdmas.values():
            rdma.wait_send()

    return pl.pallas_call(
        body,
        out_shape=jax.ShapeDtypeStruct((M, N), jnp.bfloat16),
        in_specs=[pl.BlockSpec(memory_space=pltpu.VMEM)],
        out_specs=pl.BlockSpec(memory_space=pltpu.VMEM),
        scratch_shapes=[
            pltpu.VMEM((STREAMS, N_DEV - 1, SUB, N), jnp.bfloat16),
            pltpu.VMEM((STREAMS, N_DEV - 1, SUB, N), jnp.bfloat16),
            pltpu.SemaphoreType.DMA((STREAMS, N_HOPS)),
            pltpu.SemaphoreType.DMA((STREAMS, N_HOPS)),
            pltpu.SemaphoreType.DMA((STREAMS, N_HOPS)),
            pltpu.SemaphoreType.DMA((STREAMS, N_HOPS)),
        ],
        compiler_params=pltpu.CompilerParams(
            collective_id=0,
            vmem_limit_bytes=64 * 1024 * 1024,
        ),
    )(x)


# device time: 163258 ns/iter; 1.1119x vs baseline; 1.0446x over previous
import jax
import jax.numpy as jnp
from jax import lax
from jax.experimental import pallas as pl
from jax.experimental.pallas import tpu as pltpu

N_DEV = 4
M, N = 4096, 2048
HALF = M // 2
CHUNK = HALF // N_DEV
STREAMS = 2
SUB = CHUNK // STREAMS
N_HOPS = 2 * (N_DEV - 1)


def kernel(x):

    def body(x_ref, out_ref, xs, comm_cw, comm_ccw, dma_sems,
             ssem_cw, rsem_cw, ssem_ccw, rsem_ccw):
        my = lax.axis_index("i")
        right = lax.rem(my + 1, N_DEV)
        left = lax.rem(my + N_DEV - 1, N_DEV)

        barrier = pltpu.get_barrier_semaphore()
        for nbr in (left, right):
            pl.semaphore_signal(
                barrier, inc=1,
                device_id=(nbr,), device_id_type=pl.DeviceIdType.MESH,
            )
        pl.semaphore_wait(barrier, 2)

        def subrows(base, c, j):
            return pl.ds(base + c * CHUNK + j * SUB, SUB)

        rings = (
            (1, 0, comm_cw, ssem_cw, rsem_cw, right),
            (-1, HALF, comm_ccw, ssem_ccw, rsem_ccw, left),
        )

        def rs_send(ring, j, s):
            dirn, base, comm, ssem, rsem, tgt = ring
            c = lax.rem(my - dirn * s + 2 * N_DEV, N_DEV)
            rdma = pltpu.make_async_remote_copy(
                src_ref=out_ref.at[subrows(base, c, j), :],
                dst_ref=comm.at[j, s],
                send_sem=ssem.at[j, s],
                recv_sem=rsem.at[j, s],
                device_id=(tgt,),
                device_id_type=pl.DeviceIdType.MESH,
            )
            rdma.start()
            return rdma

        def ag_send(ring, j, t):
            dirn, base, comm, ssem, rsem, tgt = ring
            h = (N_DEV - 1) + t
            c = lax.rem(my + dirn * (1 - t) + 2 * N_DEV, N_DEV)
            rdma = pltpu.make_async_remote_copy(
                src_ref=out_ref.at[subrows(base, c, j), :],
                dst_ref=out_ref.at[subrows(base, c, j), :],
                send_sem=ssem.at[j, h],
                recv_sem=rsem.at[j, h],
                device_id=(tgt,),
                device_id_type=pl.DeviceIdType.MESH,
            )
            rdma.start()
            return rdma

        loads = []
        for k in range(N_DEV):
            for ri, ring in enumerate(rings):
                dirn, base = ring[0], ring[1]
                c = lax.rem(my - dirn * k + 2 * N_DEV, N_DEV)
                slot = 2 * k + ri
                cp = pltpu.make_async_copy(
                    x_ref.at[0, pl.ds(base + c * CHUNK, CHUNK), :],
                    xs.at[slot],
                    dma_sems.at[slot],
                )
                cp.start()
                loads.append((cp, ri, c, slot))

        rdmas = {}
        for cp, ri, c, slot in loads[:2]:
            cp.wait()
            ring = rings[ri]
            base = ring[1]
            out_ref[pl.ds(base + c * CHUNK, CHUNK), :] = (
                xs[slot].astype(jnp.bfloat16))
            for j in range(STREAMS):
                rdmas[(ri, j, 0)] = rs_send(ring, j, 0)
        for cp, ri, c, slot in loads[2:]:
            cp.wait()
            base = rings[ri][1]
            out_ref[pl.ds(base + c * CHUNK, CHUNK), :] = (
                xs[slot].astype(jnp.bfloat16))

        for s in range(N_DEV - 1):
            for j in range(STREAMS):
                for ri, ring in enumerate(rings):
                    dirn, base, comm, _, _, _ = ring
                    rdmas[(ri, j, s)].wait_recv()
                    c = lax.rem(my - dirn * (s + 1) + 2 * N_DEV, N_DEV)
                    r = subrows(base, c, j)
                    out_ref[r, :] = out_ref[r, :] + comm[j, s]
                    if s < N_DEV - 2:
                        rdmas[(ri, j, s + 1)] = rs_send(ring, j, s + 1)
                    else:
                        rdmas[(ri, j, s + 1)] = ag_send(ring, j, 0)

        for t in range(N_DEV - 1):
            h = (N_DEV - 1) + t
            for j in range(STREAMS):
                for ri, ring in enumerate(rings):
                    rdmas[(ri, j, h)].wait_recv()
                    if t < N_DEV - 2:
                        rdmas[(ri, j, h + 1)] = ag_send(ring, j, t + 1)

        for rdma in rdmas.values():
            rdma.wait_send()

    return pl.pallas_call(
        body,
        out_shape=jax.ShapeDtypeStruct((M, N), jnp.bfloat16),
        in_specs=[pl.BlockSpec(memory_space=pl.ANY)],
        out_specs=pl.BlockSpec(memory_space=pltpu.VMEM),
        scratch_shapes=[
            pltpu.VMEM((2 * N_DEV, CHUNK, N), jnp.float32),
            pltpu.VMEM((STREAMS, N_DEV - 1, SUB, N), jnp.bfloat16),
            pltpu.VMEM((STREAMS, N_DEV - 1, SUB, N), jnp.bfloat16),
            pltpu.SemaphoreType.DMA((2 * N_DEV,)),
            pltpu.SemaphoreType.DMA((STREAMS, N_HOPS)),
            pltpu.SemaphoreType.DMA((STREAMS, N_HOPS)),
            pltpu.SemaphoreType.DMA((STREAMS, N_HOPS)),
            pltpu.SemaphoreType.DMA((STREAMS, N_HOPS)),
        ],
        compiler_params=pltpu.CompilerParams(
            collective_id=0,
            vmem_limit_bytes=64 * 1024 * 1024,
        ),
    )(x)


# device time: 158641 ns/iter; 1.1443x vs baseline; 1.0291x over previous
import jax
import jax.numpy as jnp
from jax import lax
from jax.experimental import pallas as pl
from jax.experimental.pallas import tpu as pltpu

N_DEV = 4
M, N = 4096, 2048
HALF = M // 2
CHUNK = HALF // N_DEV
STREAMS = 2
SUB = CHUNK // STREAMS
N_HOPS = 2 * (N_DEV - 1)


def kernel(x):

    def body(x_ref, out_ref, acc, xs, comm_cw, comm_ccw, dma_sems, wb_sems,
             ssem_cw, rsem_cw, ssem_ccw, rsem_ccw):
        my = lax.axis_index("i")
        right = lax.rem(my + 1, N_DEV)
        left = lax.rem(my + N_DEV - 1, N_DEV)

        barrier = pltpu.get_barrier_semaphore()
        for nbr in (left, right):
            pl.semaphore_signal(
                barrier, inc=1,
                device_id=(nbr,), device_id_type=pl.DeviceIdType.MESH,
            )
        pl.semaphore_wait(barrier, 2)

        def subrows(base, c, j):
            return pl.ds(base + c * CHUNK + j * SUB, SUB)

        rings = (
            (1, 0, comm_cw, ssem_cw, rsem_cw, right),
            (-1, HALF, comm_ccw, ssem_ccw, rsem_ccw, left),
        )

        def rs_send(ring, j, s):
            dirn, base, comm, ssem, rsem, tgt = ring
            c = lax.rem(my - dirn * s + 2 * N_DEV, N_DEV)
            rdma = pltpu.make_async_remote_copy(
                src_ref=acc.at[subrows(base, c, j), :],
                dst_ref=comm.at[j, s],
                send_sem=ssem.at[j, s],
                recv_sem=rsem.at[j, s],
                device_id=(tgt,),
                device_id_type=pl.DeviceIdType.MESH,
            )
            rdma.start()
            return rdma

        def ag_send(ring, j, t):
            dirn, base, comm, ssem, rsem, tgt = ring
            h = (N_DEV - 1) + t
            c = lax.rem(my + dirn * (1 - t) + 2 * N_DEV, N_DEV)
            rdma = pltpu.make_async_remote_copy(
                src_ref=acc.at[subrows(base, c, j), :],
                dst_ref=acc.at[subrows(base, c, j), :],
                send_sem=ssem.at[j, h],
                recv_sem=rsem.at[j, h],
                device_id=(tgt,),
                device_id_type=pl.DeviceIdType.MESH,
            )
            rdma.start()
            return rdma

        wb = []

        def writeback(ri, j, c, slot):
            base = rings[ri][1]
            r = subrows(base, c, j)
            cp = pltpu.make_async_copy(
                acc.at[r, :], out_ref.at[r, :],
                wb_sems.at[ri, j, slot],
            )
            cp.start()
            wb.append(cp)

        loads = []
        for k in range(N_DEV):
            for ri, ring in enumerate(rings):
                dirn, base = ring[0], ring[1]
                c = lax.rem(my - dirn * k + 2 * N_DEV, N_DEV)
                slot = 2 * k + ri
                cp = pltpu.make_async_copy(
                    x_ref.at[0, pl.ds(base + c * CHUNK, CHUNK), :],
                    xs.at[slot],
                    dma_sems.at[slot],
                )
                cp.start()
                loads.append((cp, ri, c, slot))

        rdmas = {}
        for cp, ri, c, slot in loads[:2]:
            cp.wait()
            ring = rings[ri]
            base = ring[1]
            acc[pl.ds(base + c * CHUNK, CHUNK), :] = (
                xs[slot].astype(jnp.bfloat16))
            for j in range(STREAMS):
                rdmas[(ri, j, 0)] = rs_send(ring, j, 0)
        for cp, ri, c, slot in loads[2:]:
            cp.wait()
            base = rings[ri][1]
            acc[pl.ds(base + c * CHUNK, CHUNK), :] = (
                xs[slot].astype(jnp.bfloat16))

        for s in range(N_DEV - 1):
            for j in range(STREAMS):
                for ri, ring in enumerate(rings):
                    dirn, base, comm = ring[0], ring[1], ring[2]
                    rdmas[(ri, j, s)].wait_recv()
                    c = lax.rem(my - dirn * (s + 1) + 2 * N_DEV, N_DEV)
                    r = subrows(base, c, j)
                    acc[r, :] = acc[r, :] + comm[j, s]
                    if s < N_DEV - 2:
                        rdmas[(ri, j, s + 1)] = rs_send(ring, j, s + 1)
                    else:
                        rdmas[(ri, j, s + 1)] = ag_send(ring, j, 0)
                        writeback(ri, j, c, 0)

        for t in range(N_DEV - 1):
            h = (N_DEV - 1) + t
            for j in range(STREAMS):
                for ri, ring in enumerate(rings):
                    dirn = ring[0]
                    rdmas[(ri, j, h)].wait_recv()
                    if t < N_DEV - 2:
                        rdmas[(ri, j, h + 1)] = ag_send(ring, j, t + 1)
                    writeback(ri, j,
                              lax.rem(my - dirn * t + 2 * N_DEV, N_DEV),
                              1 + t)

        for rdma in rdmas.values():
            rdma.wait_send()
        for cp in wb:
            cp.wait()

    return pl.pallas_call(
        body,
        out_shape=jax.ShapeDtypeStruct((M, N), jnp.bfloat16),
        in_specs=[pl.BlockSpec(memory_space=pl.ANY)],
        out_specs=pl.BlockSpec(memory_space=pl.ANY),
        scratch_shapes=[
            pltpu.VMEM((M, N), jnp.bfloat16),
            pltpu.VMEM((2 * N_DEV, CHUNK, N), jnp.float32),
            pltpu.VMEM((STREAMS, N_DEV - 1, SUB, N), jnp.bfloat16),
            pltpu.VMEM((STREAMS, N_DEV - 1, SUB, N), jnp.bfloat16),
            pltpu.SemaphoreType.DMA((2 * N_DEV,)),
            pltpu.SemaphoreType.DMA((2, STREAMS, N_DEV)),
            pltpu.SemaphoreType.DMA((STREAMS, N_HOPS)),
            pltpu.SemaphoreType.DMA((STREAMS, N_HOPS)),
            pltpu.SemaphoreType.DMA((STREAMS, N_HOPS)),
            pltpu.SemaphoreType.DMA((STREAMS, N_HOPS)),
        ],
        compiler_params=pltpu.CompilerParams(
            collective_id=0,
            vmem_limit_bytes=64 * 1024 * 1024,
        ),
    )(x)


# device time: 156184 ns/iter; 1.1623x vs baseline; 1.0157x over previous
import jax
import jax.numpy as jnp
from jax import lax
from jax.experimental import pallas as pl
from jax.experimental.pallas import tpu as pltpu

N_DEV = 4
M, N = 4096, 2048
HALF = M // 2
CHUNK = HALF // N_DEV
STREAMS = 2
SUB = CHUNK // STREAMS
N_HOPS = 2 * (N_DEV - 1)


def kernel(x):

    def body(x_ref, out_ref, acc, xs, comm_cw, comm_ccw,
             dma_sems, wb_sems,
             ssem_cw, rsem_cw, ssem_ccw, rsem_ccw):
        my = lax.axis_index("i")
        right = lax.rem(my + 1, N_DEV)
        left = lax.rem(my + N_DEV - 1, N_DEV)

        def subrows(base, c, j):
            return pl.ds(base + c * CHUNK + j * SUB, SUB)

        rings = (
            (1, 0, comm_cw, ssem_cw, rsem_cw, right),
            (-1, HALF, comm_ccw, ssem_ccw, rsem_ccw, left),
        )

        def rs_send(ring, j, s):
            dirn, base, comm, ssem, rsem, tgt = ring
            c = lax.rem(my - dirn * s + 2 * N_DEV, N_DEV)
            rdma = pltpu.make_async_remote_copy(
                src_ref=acc.at[subrows(base, c, j), :],
                dst_ref=comm.at[j, s],
                send_sem=ssem.at[j, s],
                recv_sem=rsem.at[j, s],
                device_id=(tgt,),
                device_id_type=pl.DeviceIdType.MESH,
            )
            rdma.start()
            return rdma

        def ag_send(ring, j, t):
            dirn, base, comm, ssem, rsem, tgt = ring
            h = (N_DEV - 1) + t
            c = lax.rem(my + dirn * (1 - t) + 2 * N_DEV, N_DEV)
            rdma = pltpu.make_async_remote_copy(
                src_ref=acc.at[subrows(base, c, j), :],
                dst_ref=acc.at[subrows(base, c, j), :],
                send_sem=ssem.at[j, h],
                recv_sem=rsem.at[j, h],
                device_id=(tgt,),
                device_id_type=pl.DeviceIdType.MESH,
            )
            rdma.start()
            return rdma

        wb = []

        def writeback(ri, j, c, slot):
            base = rings[ri][1]
            r = subrows(base, c, j)
            cp = pltpu.make_async_copy(
                acc.at[r, :], out_ref.at[r, :],
                wb_sems.at[ri, j, slot],
            )
            cp.start()
            wb.append(cp)

        first_loads = []
        for j in range(STREAMS):
            for ri, ring in enumerate(rings):
                base = ring[1]
                r = subrows(base, my, j)
                cp = pltpu.make_async_copy(
                    x_ref.at[0, r, :],
                    xs.at[ri, pl.ds(j * SUB, SUB)],
                    dma_sems.at[STREAMS * ri + j],
                )
                cp.start()
                first_loads.append((cp, ri, j, r))
        loads = []
        for k in range(1, N_DEV):
            for ri, ring in enumerate(rings):
                dirn, base = ring[0], ring[1]
                c = lax.rem(my - dirn * k + 2 * N_DEV, N_DEV)
                slot = 2 * k + ri
                cp = pltpu.make_async_copy(
                    x_ref.at[0, pl.ds(base + c * CHUNK, CHUNK), :],
                    xs.at[slot],
                    dma_sems.at[2 + slot],
                )
                cp.start()
                loads.append((cp, ri, c, slot))

        barrier = pltpu.get_barrier_semaphore()
        for nbr in (left, right):
            pl.semaphore_signal(
                barrier, inc=1,
                device_id=(nbr,), device_id_type=pl.DeviceIdType.MESH,
            )
        pl.semaphore_wait(barrier, 2)

        rdmas = {}
        for cp, ri, j, r in first_loads:
            cp.wait()
            acc[r, :] = xs[ri, pl.ds(j * SUB, SUB)].astype(jnp.bfloat16)
            rdmas[(ri, j, 0)] = rs_send(rings[ri], j, 0)
        for cp, ri, c, slot in loads:
            cp.wait()
            base = rings[ri][1]
            acc[pl.ds(base + c * CHUNK, CHUNK), :] = (
                xs[slot].astype(jnp.bfloat16))

        for s in range(N_DEV - 1):
            for j in range(STREAMS):
                for ri, ring in enumerate(rings):
                    dirn, base, comm = ring[0], ring[1], ring[2]
                    rdmas[(ri, j, s)].wait_recv()
                    c = lax.rem(my - dirn * (s + 1) + 2 * N_DEV, N_DEV)
                    r = subrows(base, c, j)
                    acc[r, :] = acc[r, :] + comm[j, s]
                    if s < N_DEV - 2:
                        rdmas[(ri, j, s + 1)] = rs_send(ring, j, s + 1)
                    else:
                        rdmas[(ri, j, s + 1)] = ag_send(ring, j, 0)
                        writeback(ri, j, c, 0)

        for t in range(N_DEV - 1):
            h = (N_DEV - 1) + t
            for j in range(STREAMS):
                for ri, ring in enumerate(rings):
                    dirn = ring[0]
                    rdmas[(ri, j, h)].wait_recv()
                    if t < N_DEV - 2:
                        rdmas[(ri, j, h + 1)] = ag_send(ring, j, t + 1)
                    writeback(ri, j,
                              lax.rem(my - dirn * t + 2 * N_DEV, N_DEV),
                              1 + t)

        for rdma in rdmas.values():
            rdma.wait_send()
        for cp in wb:
            cp.wait()

    return pl.pallas_call(
        body,
        out_shape=jax.ShapeDtypeStruct((M, N), jnp.bfloat16),
        in_specs=[pl.BlockSpec(memory_space=pl.ANY)],
        out_specs=pl.BlockSpec(memory_space=pl.ANY),
        scratch_shapes=[
            pltpu.VMEM((M, N), jnp.bfloat16),
            pltpu.VMEM((2 * N_DEV, CHUNK, N), jnp.float32),
            pltpu.VMEM((STREAMS, N_DEV - 1, SUB, N), jnp.bfloat16),
            pltpu.VMEM((STREAMS, N_DEV - 1, SUB, N), jnp.bfloat16),
            pltpu.SemaphoreType.DMA((2 * N_DEV + 2,)),
            pltpu.SemaphoreType.DMA((2, STREAMS, N_DEV)),
            pltpu.SemaphoreType.DMA((STREAMS, N_HOPS)),
            pltpu.SemaphoreType.DMA((STREAMS, N_HOPS)),
            pltpu.SemaphoreType.DMA((STREAMS, N_HOPS)),
            pltpu.SemaphoreType.DMA((STREAMS, N_HOPS)),
        ],
        compiler_params=pltpu.CompilerParams(
            collective_id=0,
            vmem_limit_bytes=64 * 1024 * 1024,
        ),
    )(x)


# device time: 155886 ns/iter; 1.1645x vs baseline; 1.0019x over previous
import jax
import jax.numpy as jnp
from jax import lax
from jax.experimental import pallas as pl
from jax.experimental.pallas import tpu as pltpu

N_DEV = 4
M, N = 4096, 2048
HALF = M // 2
CHUNK = HALF // N_DEV
STREAMS = 4
SUB = CHUNK // STREAMS
N_HOPS = 2 * (N_DEV - 1)


def kernel(x):

    def body(x_ref, out_ref, acc, xs, comm_cw, comm_ccw,
             dma_sems, wb_sems,
             ssem_cw, rsem_cw, ssem_ccw, rsem_ccw):
        my = lax.axis_index("i")
        right = lax.rem(my + 1, N_DEV)
        left = lax.rem(my + N_DEV - 1, N_DEV)

        def subrows(base, c, j):
            return pl.ds(base + c * CHUNK + j * SUB, SUB)

        rings = (
            (1, 0, comm_cw, ssem_cw, rsem_cw, right),
            (-1, HALF, comm_ccw, ssem_ccw, rsem_ccw, left),
        )

        def rs_send(ring, j, s):
            dirn, base, comm, ssem, rsem, tgt = ring
            c = lax.rem(my - dirn * s + 2 * N_DEV, N_DEV)
            rdma = pltpu.make_async_remote_copy(
                src_ref=acc.at[subrows(base, c, j), :],
                dst_ref=comm.at[j, s],
                send_sem=ssem.at[j, s],
                recv_sem=rsem.at[j, s],
                device_id=(tgt,),
                device_id_type=pl.DeviceIdType.MESH,
            )
            rdma.start()
            return rdma

        def ag_send(ring, j, t):
            dirn, base, comm, ssem, rsem, tgt = ring
            h = (N_DEV - 1) + t
            c = lax.rem(my + dirn * (1 - t) + 2 * N_DEV, N_DEV)
            rdma = pltpu.make_async_remote_copy(
                src_ref=acc.at[subrows(base, c, j), :],
                dst_ref=acc.at[subrows(base, c, j), :],
                send_sem=ssem.at[j, h],
                recv_sem=rsem.at[j, h],
                device_id=(tgt,),
                device_id_type=pl.DeviceIdType.MESH,
            )
            rdma.start()
            return rdma

        wb = []

        def writeback(ri, j, c, slot):
            base = rings[ri][1]
            r = subrows(base, c, j)
            cp = pltpu.make_async_copy(
                acc.at[r, :], out_ref.at[r, :],
                wb_sems.at[ri, j, slot],
            )
            cp.start()
            wb.append(cp)

        first_loads = []
        for j in range(STREAMS):
            for ri, ring in enumerate(rings):
                base = ring[1]
                r = subrows(base, my, j)
                cp = pltpu.make_async_copy(
                    x_ref.at[0, r, :],
                    xs.at[ri, pl.ds(j * SUB, SUB)],
                    dma_sems.at[ri * STREAMS + j],
                )
                cp.start()
                first_loads.append((cp, ri, j, r))
        loads = []
        for k in range(1, N_DEV):
            for ri, ring in enumerate(rings):
                dirn, base = ring[0], ring[1]
                c = lax.rem(my - dirn * k + 2 * N_DEV, N_DEV)
                slot = 2 * k + ri
                cp = pltpu.make_async_copy(
                    x_ref.at[0, pl.ds(base + c * CHUNK, CHUNK), :],
                    xs.at[slot],
                    dma_sems.at[2 * STREAMS + slot - 2],
                )
                cp.start()
                loads.append((cp, ri, c, slot))

        barrier = pltpu.get_barrier_semaphore()
        for nbr in (left, right):
            pl.semaphore_signal(
                barrier, inc=1,
                device_id=(nbr,), device_id_type=pl.DeviceIdType.MESH,
            )
        pl.semaphore_wait(barrier, 2)

        rdmas = {}
        for cp, ri, j, r in first_loads:
            cp.wait()
            acc[r, :] = xs[ri, pl.ds(j * SUB, SUB)].astype(jnp.bfloat16)
            rdmas[(ri, j, 0)] = rs_send(rings[ri], j, 0)
        for cp, ri, c, slot in loads:
            cp.wait()
            base = rings[ri][1]
            acc[pl.ds(base + c * CHUNK, CHUNK), :] = (
                xs[slot].astype(jnp.bfloat16))

        for s in range(N_DEV - 1):
            for j in range(STREAMS):
                for ri, ring in enumerate(rings):
                    dirn, base, comm = ring[0], ring[1], ring[2]
                    rdmas[(ri, j, s)].wait_recv()
                    c = lax.rem(my - dirn * (s + 1) + 2 * N_DEV, N_DEV)
                    r = subrows(base, c, j)
                    acc[r, :] = acc[r, :] + comm[j, s]
                    if s < N_DEV - 2:
                        rdmas[(ri, j, s + 1)] = rs_send(ring, j, s + 1)
                    else:
                        rdmas[(ri, j, s + 1)] = ag_send(ring, j, 0)
                        writeback(ri, j, c, 0)

        for t in range(N_DEV - 1):
            h = (N_DEV - 1) + t
            for j in range(STREAMS):
                for ri, ring in enumerate(rings):
                    dirn = ring[0]
                    rdmas[(ri, j, h)].wait_recv()
                    if t < N_DEV - 2:
                        rdmas[(ri, j, h + 1)] = ag_send(ring, j, t + 1)
                    writeback(ri, j,
                              lax.rem(my - dirn * t + 2 * N_DEV, N_DEV),
                              1 + t)

        for rdma in rdmas.values():
            rdma.wait_send()
        for cp in wb:
            cp.wait()

    return pl.pallas_call(
        body,
        out_shape=jax.ShapeDtypeStruct((M, N), jnp.bfloat16),
        in_specs=[pl.BlockSpec(memory_space=pl.ANY)],
        out_specs=pl.BlockSpec(memory_space=pl.ANY),
        scratch_shapes=[
            pltpu.VMEM((M, N), jnp.bfloat16),
            pltpu.VMEM((2 * N_DEV, CHUNK, N), jnp.float32),
            pltpu.VMEM((STREAMS, N_DEV - 1, SUB, N), jnp.bfloat16),
            pltpu.VMEM((STREAMS, N_DEV - 1, SUB, N), jnp.bfloat16),
            pltpu.SemaphoreType.DMA((2 * STREAMS + 2 * (N_DEV - 1),)),
            pltpu.SemaphoreType.DMA((2, STREAMS, N_DEV)),
            pltpu.SemaphoreType.DMA((STREAMS, N_HOPS)),
            pltpu.SemaphoreType.DMA((STREAMS, N_HOPS)),
            pltpu.SemaphoreType.DMA((STREAMS, N_HOPS)),
            pltpu.SemaphoreType.DMA((STREAMS, N_HOPS)),
        ],
        compiler_params=pltpu.CompilerParams(
            collective_id=0,
            vmem_limit_bytes=64 * 1024 * 1024,
        ),
    )(x)
